# baseline (speedup 1.0000x reference)
import functools
import math

import jax
import jax.numpy as jnp
from jax.experimental import pallas as pl
from jax.experimental.pallas import tpu as pltpu

_GUIDE_SIGMA = 0.2
_B, _N_MAX, _T_MAX = 16, 512, 2048
_RH = 128
_NQ = _N_MAX // _RH
_CT = 128
_NSLOTS = 12
_INV_TOTAL = 1.0 / float(_B * _N_MAX * _T_MAX)
_SCALE = math.sqrt(math.log2(math.e) / (2.0 * _GUIDE_SIGMA**2))


def _body(info_ref, al_ref, out_ref, bufs_ref, acc_ref, sems_ref):
    b = pl.program_id(0)

    def quarter_copy(batch, q):
        slot = (4 * batch + q) % _NSLOTS
        return pltpu.make_async_copy(
            al_ref.at[batch, pl.ds(q * _RH, _RH), :],
            bufs_ref.at[slot],
            sems_ref.at[slot],
        )

    def for_each_quarter(batch, fn):
        n_len = info_ref[1, batch]
        for q in range(_NQ):
            if q == 0:
                fn(batch, q)
            else:
                pl.when(q * _RH < n_len)(lambda q=q: fn(batch, q))

    def issue(batch):
        for_each_quarter(batch, lambda bt, q: quarter_copy(bt, q).start())

    def wait(batch):
        for_each_quarter(batch, lambda bt, q: quarter_copy(bt, q).wait())

    @pl.when(b == 0)
    def _prologue():
        acc_ref[...] = jnp.zeros((_RH, _CT), jnp.float32)
        issue(jnp.int32(0))
        issue(jnp.int32(1))
        issue(jnp.int32(2))

    @pl.when((b > 0) & (b + 2 < _B))
    def _lookahead():
        issue(b + 2)

    wait(b)

    n_len = info_ref[1, b]
    nf = n_len.astype(jnp.float32)
    tf = info_ref[2, b].astype(jnp.float32)
    t_chunks = info_ref[3, b]

    inv_n = 1.0 / nf
    ratio = nf / tf
    scaled_inv_n = inv_n * _SCALE

    tbase = jax.lax.broadcasted_iota(jnp.int32, (1, _CT), 1).astype(jnp.float32)

    for q in range(_NQ):
        slot = (4 * b + q) % _NSLOTS

        def quarter(q=q, slot=slot):
            ccol = (
                jax.lax.broadcasted_iota(jnp.int32, (_RH, 1), 0).astype(
                    jnp.float32
                )
                + float(q * _RH)
            )
            c2 = ccol * scaled_inv_n

            def guide(k, masked):
                trow = tbase + (k * _CT).astype(jnp.float32)
                o2 = jnp.floor(ratio * trow) * scaled_inv_n
                x = c2 - o2
                negx = o2 - c2
                u = x * negx
                if masked:
                    tmf = jnp.where(trow < tf, 1.0, 0.0)
                    u = u * tmf
                al = bufs_ref[slot, :, pl.ds(k * _CT, _CT)]
                return al * (1.0 - jnp.exp2(u))

            def chunk_pair(i, acc):
                acc = acc + guide(2 * i, masked=False)
                return acc + guide(2 * i + 1, masked=False)

            t_even = ((t_chunks - 1) // 2) * 2
            acc = jax.lax.fori_loop(
                0, t_even // 2, chunk_pair, jnp.zeros((_RH, _CT), jnp.float32)
            )

            def chunk_tail(k, acc):
                return acc + guide(k, masked=True)

            acc = jax.lax.fori_loop(t_even, t_chunks, chunk_tail, acc)
            cmask = ccol < nf
            acc_ref[...] += jnp.where(cmask, acc, 0.0)

        if q == 0:
            quarter()
        else:
            pl.when(q * _RH < n_len)(quarter)

    @pl.when(b == _B - 1)
    def _finish():
        out_ref[0, 0] = jnp.sum(acc_ref[...]) * _INV_TOTAL


@functools.partial(jax.jit, static_argnames=())
def kernel(alignments, input_lengths, target_lengths):
    n_i = input_lengths.astype(jnp.int32)
    t_i = target_lengths.astype(jnp.int32)
    n_quarters = (n_i + (_RH - 1)) // _RH
    t_chunks = (t_i + (_CT - 1)) // _CT
    info = jnp.stack([n_quarters, n_i, t_i, t_chunks])

    grid_spec = pltpu.PrefetchScalarGridSpec(
        num_scalar_prefetch=1,
        grid=(_B,),
        in_specs=[pl.BlockSpec(memory_space=pl.ANY)],
        out_specs=pl.BlockSpec(
            (1, 1), lambda b, info: (0, 0), memory_space=pltpu.SMEM
        ),
        scratch_shapes=[
            pltpu.VMEM((_NSLOTS, _RH, _T_MAX), jnp.float32),
            pltpu.VMEM((_RH, _CT), jnp.float32),
            pltpu.SemaphoreType.DMA((_NSLOTS,)),
        ],
    )

    out = pl.pallas_call(
        _body,
        grid_spec=grid_spec,
        out_shape=jax.ShapeDtypeStruct((1, 1), jnp.float32),
        compiler_params=pltpu.CompilerParams(
            dimension_semantics=("arbitrary",),
        ),
    )(info, alignments)
    return out[0, 0]


# speedup vs baseline: 1.0106x; 1.0106x over previous
import functools
import math

import jax
import jax.numpy as jnp
from jax.experimental import pallas as pl
from jax.experimental.pallas import tpu as pltpu

_GUIDE_SIGMA = 0.2
_B, _N_MAX, _T_MAX = 16, 512, 2048
_PH = 64
_NP = _N_MAX // _PH
_RH = 128
_NQ = _N_MAX // _RH
_CT = 128
_NREG = 3
_INV_TOTAL = 1.0 / float(_B * _N_MAX * _T_MAX)
_SCALE = math.sqrt(math.log2(math.e) / (2.0 * _GUIDE_SIGMA**2))


def _body(info_ref, al_ref, out_ref, bufs_ref, acc_ref, sems_ref):
    b = pl.program_id(0)

    def piece_copy(batch, p):
        region = batch % _NREG
        return pltpu.make_async_copy(
            al_ref.at[batch, pl.ds(p * _PH, _PH), :],
            bufs_ref.at[region, pl.ds(p * _PH, _PH), :],
            sems_ref.at[region, p],
        )

    def for_each_piece(batch, fn):
        n_len = info_ref[1, batch]
        for p in range(_NP):
            if p == 0:
                fn(batch, p)
            else:
                pl.when(p * _PH < n_len)(lambda p=p: fn(batch, p))

    def issue(batch):
        for_each_piece(batch, lambda bt, p: piece_copy(bt, p).start())

    def wait(batch):
        for_each_piece(batch, lambda bt, p: piece_copy(bt, p).wait())

    @pl.when(b == 0)
    def _prologue():
        acc_ref[...] = jnp.zeros((_RH, _CT), jnp.float32)
        issue(jnp.int32(0))
        issue(jnp.int32(1))
        issue(jnp.int32(2))

    @pl.when((b > 0) & (b + 2 < _B))
    def _lookahead():
        issue(b + 2)

    wait(b)

    region = b % _NREG
    n_len = info_ref[1, b]
    nf = n_len.astype(jnp.float32)
    tf = info_ref[2, b].astype(jnp.float32)
    t_chunks = info_ref[3, b]

    inv_n = 1.0 / nf
    ratio = nf / tf
    scaled_inv_n = inv_n * _SCALE

    tbase = jax.lax.broadcasted_iota(jnp.int32, (1, _CT), 1).astype(jnp.float32)

    for q in range(_NQ):

        def quarter(q=q):
            ccol = (
                jax.lax.broadcasted_iota(jnp.int32, (_RH, 1), 0).astype(
                    jnp.float32
                )
                + float(q * _RH)
            )
            c2 = ccol * scaled_inv_n

            def guide(k, masked):
                trow = tbase + (k * _CT).astype(jnp.float32)
                o2 = jnp.floor(ratio * trow) * scaled_inv_n
                x = c2 - o2
                negx = o2 - c2
                u = x * negx
                if masked:
                    tmf = jnp.where(trow < tf, 1.0, 0.0)
                    u = u * tmf
                al = bufs_ref[
                    region, q * _RH : (q + 1) * _RH, pl.ds(k * _CT, _CT)
                ]
                return al * (1.0 - jnp.exp2(u))

            def chunk_pair(i, acc):
                acc = acc + guide(2 * i, masked=False)
                return acc + guide(2 * i + 1, masked=False)

            t_even = ((t_chunks - 1) // 2) * 2
            acc = jax.lax.fori_loop(
                0, t_even // 2, chunk_pair, jnp.zeros((_RH, _CT), jnp.float32)
            )

            def chunk_tail(k, acc):
                return acc + guide(k, masked=True)

            acc = jax.lax.fori_loop(t_even, t_chunks, chunk_tail, acc)
            cmask = ccol < nf
            acc_ref[...] += jnp.where(cmask, acc, 0.0)

        if q == 0:
            quarter()
        else:
            pl.when(q * _RH < n_len)(quarter)

    @pl.when(b == _B - 1)
    def _finish():
        out_ref[0, 0] = jnp.sum(acc_ref[...]) * _INV_TOTAL


@functools.partial(jax.jit, static_argnames=())
def kernel(alignments, input_lengths, target_lengths):
    n_i = input_lengths.astype(jnp.int32)
    t_i = target_lengths.astype(jnp.int32)
    n_pieces = (n_i + (_PH - 1)) // _PH
    t_chunks = (t_i + (_CT - 1)) // _CT
    info = jnp.stack([n_pieces, n_i, t_i, t_chunks])

    grid_spec = pltpu.PrefetchScalarGridSpec(
        num_scalar_prefetch=1,
        grid=(_B,),
        in_specs=[pl.BlockSpec(memory_space=pl.ANY)],
        out_specs=pl.BlockSpec(
            (1, 1), lambda b, info: (0, 0), memory_space=pltpu.SMEM
        ),
        scratch_shapes=[
            pltpu.VMEM((_NREG, _N_MAX, _T_MAX), jnp.float32),
            pltpu.VMEM((_RH, _CT), jnp.float32),
            pltpu.SemaphoreType.DMA((_NREG, _NP)),
        ],
    )

    out = pl.pallas_call(
        _body,
        grid_spec=grid_spec,
        out_shape=jax.ShapeDtypeStruct((1, 1), jnp.float32),
        compiler_params=pltpu.CompilerParams(
            dimension_semantics=("arbitrary",),
        ),
    )(info, alignments)
    return out[0, 0]


# speedup vs baseline: 1.1360x; 1.1240x over previous
import functools
import math

import jax
import jax.numpy as jnp
from jax.experimental import pallas as pl
from jax.experimental.pallas import tpu as pltpu

_GUIDE_SIGMA = 0.2
_B, _N_MAX, _T_MAX = 16, 512, 2048
_PH = 64
_NP = _N_MAX // _PH
_RH = 128
_NQ = _N_MAX // _RH
_CT = 128
_NREG = 4
_INV_TOTAL = 1.0 / float(_B * _N_MAX * _T_MAX)
_SCALE = math.sqrt(math.log2(math.e) / (2.0 * _GUIDE_SIGMA**2))


def _body(info_ref, al_ref, out_ref, bufs_ref, acc_ref, sems_ref):
    b = pl.program_id(0)

    def piece_copy(batch, p):
        region = batch % _NREG
        return pltpu.make_async_copy(
            al_ref.at[batch, pl.ds(p * _PH, _PH), :],
            bufs_ref.at[region, pl.ds(p * _PH, _PH), :],
            sems_ref.at[region, p],
        )

    def for_each_piece(batch, fn):
        n_len = info_ref[1, batch]
        for p in range(_NP):
            if p == 0:
                fn(batch, p)
            else:
                pl.when(p * _PH < n_len)(lambda p=p: fn(batch, p))

    def issue(batch):
        for_each_piece(batch, lambda bt, p: piece_copy(bt, p).start())

    def wait(batch):
        for_each_piece(batch, lambda bt, p: piece_copy(bt, p).wait())

    @pl.when(b == 0)
    def _prologue():
        acc_ref[...] = jnp.zeros((_RH, _CT), jnp.float32)
        issue(jnp.int32(0))
        issue(jnp.int32(1))
        issue(jnp.int32(2))
        issue(jnp.int32(3))

    @pl.when((b > 0) & (b + 3 < _B))
    def _lookahead():
        issue(b + 3)

    wait(b)

    region = b % _NREG
    n_len = info_ref[1, b]
    nf = n_len.astype(jnp.float32)
    tf = info_ref[2, b].astype(jnp.float32)
    t_chunks = info_ref[3, b]

    inv_n = 1.0 / nf
    ratio = nf / tf
    scaled_inv_n = inv_n * _SCALE

    tbase = jax.lax.broadcasted_iota(jnp.int32, (1, _CT), 1).astype(jnp.float32)

    for q in range(_NQ):

        def quarter(q=q):
            ccol = (
                jax.lax.broadcasted_iota(jnp.int32, (_RH, 1), 0).astype(
                    jnp.float32
                )
                + float(q * _RH)
            )
            c2 = ccol * scaled_inv_n

            def guide(k, masked):
                trow = tbase + (k * _CT).astype(jnp.float32)
                o2 = jnp.floor(ratio * trow) * scaled_inv_n
                x = c2 - o2
                negx = o2 - c2
                u = x * negx
                if masked:
                    tmf = jnp.where(trow < tf, 1.0, 0.0)
                    u = u * tmf
                al = bufs_ref[
                    region, q * _RH : (q + 1) * _RH, pl.ds(k * _CT, _CT)
                ]
                return al * (1.0 - jnp.exp2(u))

            def chunk_pair(i, acc):
                acc = acc + guide(2 * i, masked=False)
                return acc + guide(2 * i + 1, masked=False)

            t_even = ((t_chunks - 1) // 2) * 2
            acc = jax.lax.fori_loop(
                0, t_even // 2, chunk_pair, jnp.zeros((_RH, _CT), jnp.float32)
            )

            def chunk_tail(k, acc):
                return acc + guide(k, masked=True)

            acc = jax.lax.fori_loop(t_even, t_chunks, chunk_tail, acc)
            cmask = ccol < nf
            acc_ref[...] += jnp.where(cmask, acc, 0.0)

        if q == 0:
            quarter()
        else:
            pl.when(q * _RH < n_len)(quarter)

    @pl.when(b == _B - 1)
    def _finish():
        out_ref[0, 0] = jnp.sum(acc_ref[...]) * _INV_TOTAL


@functools.partial(jax.jit, static_argnames=())
def kernel(alignments, input_lengths, target_lengths):
    n_i = input_lengths.astype(jnp.int32)
    t_i = target_lengths.astype(jnp.int32)
    n_pieces = (n_i + (_PH - 1)) // _PH
    t_chunks = (t_i + (_CT - 1)) // _CT
    info = jnp.stack([n_pieces, n_i, t_i, t_chunks])

    grid_spec = pltpu.PrefetchScalarGridSpec(
        num_scalar_prefetch=1,
        grid=(_B,),
        in_specs=[pl.BlockSpec(memory_space=pl.ANY)],
        out_specs=pl.BlockSpec(
            (1, 1), lambda b, info: (0, 0), memory_space=pltpu.SMEM
        ),
        scratch_shapes=[
            pltpu.VMEM((_NREG, _N_MAX, _T_MAX), jnp.float32),
            pltpu.VMEM((_RH, _CT), jnp.float32),
            pltpu.SemaphoreType.DMA((_NREG, _NP)),
        ],
    )

    out = pl.pallas_call(
        _body,
        grid_spec=grid_spec,
        out_shape=jax.ShapeDtypeStruct((1, 1), jnp.float32),
        compiler_params=pltpu.CompilerParams(
            dimension_semantics=("arbitrary",),
        ),
    )(info, alignments)
    return out[0, 0]
